# Initial kernel scaffold; baseline (speedup 1.0000x reference)
import jax
import jax.numpy as jnp
from jax.experimental import pallas as pl


def kernel(batch_y_hat, batch_y, batch_obj_scores, min_obj_score):
    raise NotImplementedError("write your pallas kernel here")


# speedup vs baseline: 3.3670x; 3.3670x over previous
import functools

import jax
import jax.numpy as jnp
from jax import lax
from jax.experimental import pallas as pl
from jax.experimental.pallas import tpu as pltpu
from jax.experimental.pallas import tpu_sc as plsc

_B = 8
_NP = 20000
_NW = 32
_PPW = 640
_NG = _PPW // 16
_NGT = 100
_NGT_PAD = 112


def _sc_body(yhat_hbm, obj_hbm, gt_hbm, minobj_hbm, out_hbm,
             yhat_v, obj_v, gt_v, minobj_v, out_v):
    c = lax.axis_index("c")
    s = lax.axis_index("s")
    wid = s * 2 + c

    pltpu.sync_copy(yhat_hbm.at[wid], yhat_v)
    pltpu.sync_copy(obj_hbm.at[wid], obj_v)
    pltpu.sync_copy(gt_hbm, gt_v)
    pltpu.sync_copy(minobj_hbm, minobj_v)
    minobj = minobj_v[...]

    zeros_i = jnp.zeros((16,), jnp.int32)
    iota16 = lax.iota(jnp.int32, 16)

    def batch_body(b, carry):
        gbase = b * (4 * _NGT_PAD)
        gidx0 = jnp.full((16,), gbase, jnp.int32)

        def group_body(g, accs):
            loc_acc, m_acc, xm_acc = accs
            sl = pl.ds(g * 16, 16)
            cx = yhat_v[b, 0, sl]
            cy = yhat_v[b, 1, sl]
            w = yhat_v[b, 2, sl]
            h = yhat_v[b, 3, sl]
            ob = obj_v[b, sl]
            hw = w * 0.5
            hh = h * 0.5
            l1 = cx - hw
            r1 = cx + hw
            t1 = cy - hh
            b1 = cy + hh
            a1 = (r1 - l1) * (b1 - t1)
            keep = ob > minobj

            def jg_body(jg, st):
                bi, bu, bj = st
                jo = jg * 16
                gcxv = gt_v[pl.ds(gbase + jo, 16)]
                gcyv = gt_v[pl.ds(gbase + _NGT_PAD + jo, 16)]
                gwv = gt_v[pl.ds(gbase + 2 * _NGT_PAD + jo, 16)]
                ghv = gt_v[pl.ds(gbase + 3 * _NGT_PAD + jo, 16)]
                l2v = gcxv - gwv * 0.5
                r2v = gcxv + gwv * 0.5
                t2v = gcyv - ghv * 0.5
                b2v = gcyv + ghv * 0.5
                a2v = (r2v - l2v) * (b2v - t2v)
                for je in range(16):
                    l2 = l2v[je]
                    r2 = r2v[je]
                    t2 = t2v[je]
                    b2 = b2v[je]
                    a2 = a2v[je]
                    iw = jnp.maximum(jnp.minimum(r1, r2) - jnp.maximum(l1, l2), 0.0)
                    ih = jnp.maximum(jnp.minimum(b1, b2) - jnp.maximum(t1, t2), 0.0)
                    inter = iw * ih
                    union = (a1 + a2) - inter
                    better = inter * bu > bi * union
                    bi = jnp.where(better, inter, bi)
                    bu = jnp.where(better, union, bu)
                    bj = jnp.where(better, jo + je, bj)
                return bi, bu, bj

            bi0 = jnp.zeros((16,), jnp.float32)
            bu0 = jnp.ones((16,), jnp.float32)
            bi, bu, bj = lax.fori_loop(0, _NGT_PAD // 16, jg_body,
                                       (bi0, bu0, zeros_i))

            matched = keep & (bi + bi > bu)
            gi = gidx0 + bj
            gcx = plsc.load_gather(gt_v, [gi])
            gcy = plsc.load_gather(gt_v, [gi + _NGT_PAD])
            gw = plsc.load_gather(gt_v, [gi + 2 * _NGT_PAD])
            gh = plsc.load_gather(gt_v, [gi + 3 * _NGT_PAD])
            dcx = cx - gcx
            dcy = cy - gcy
            dw = w - gw
            dh = h - gh
            d = dcx * dcx + dcy * dcy + dw * dw + dh * dh
            zf = jnp.zeros((16,), jnp.float32)
            loc_acc = loc_acc + jnp.where(matched, d, zf)
            m_acc = m_acc + jnp.where(matched, 1.0, 0.0)
            xm_acc = xm_acc + jnp.where(matched, ob, zf)
            return loc_acc, m_acc, xm_acc

        z = jnp.zeros((16,), jnp.float32)
        loc_acc, m_acc, xm_acc = lax.fori_loop(0, _NG, group_body, (z, z, z))
        obase = b * 48
        plsc.store_scatter(out_v, [obase + iota16], loc_acc)
        plsc.store_scatter(out_v, [obase + 16 + iota16], m_acc)
        plsc.store_scatter(out_v, [obase + 32 + iota16], xm_acc)
        return carry
    lax.fori_loop(0, _B, batch_body, 0)

    pltpu.sync_copy(out_v, out_hbm.at[wid])


_sc_match = pl.kernel(
    _sc_body,
    out_type=jax.ShapeDtypeStruct((_NW, _B * 3 * 16), jnp.float32),
    mesh=plsc.VectorSubcoreMesh(core_axis_name="c", subcore_axis_name="s"),
    compiler_params=pltpu.CompilerParams(needs_layout_passes=False),
    scratch_types=[
        pltpu.VMEM((_B, 4, _PPW), jnp.float32),
        pltpu.VMEM((_B, _PPW), jnp.float32),
        pltpu.VMEM((_B * 4 * _NGT_PAD,), jnp.float32),
        pltpu.VMEM((16,), jnp.float32),
        pltpu.VMEM((_B * 3 * 16,), jnp.float32),
    ],
)


def _tc_body(obj_ref, minobj_ref, part_ref, out_ref):
    x = obj_ref[...]
    minobj = minobj_ref[0, 0]
    keep = (x > minobj).astype(jnp.float32)
    k_b = jnp.sum(keep, axis=1)
    bce = jnp.maximum(x, 0.0) + jnp.log1p(jnp.exp(-jnp.abs(x)))
    a_b = jnp.sum(keep * bce, axis=1)
    part = part_ref[...].reshape(_NW, _B, 3, 16)
    sums = jnp.sum(jnp.sum(part, axis=3), axis=0)
    loc_sum = sums[:, 0]
    m = sums[:, 1]
    xm = sums[:, 2]
    loc = jnp.where(m > 0, loc_sum / (4.0 * jnp.maximum(m, 1.0)), 0.0)
    obj = (a_b - xm) / k_b
    pen = 0.1 * (k_b - m)
    total = jnp.sum(loc + obj + pen) / _B
    out_ref[...] = jnp.full((1, 1), total, jnp.float32)


_tc_combine = pl.pallas_call(
    _tc_body,
    out_shape=jax.ShapeDtypeStruct((1, 1), jnp.float32),
)


def kernel(batch_y_hat, batch_y, batch_obj_scores, min_obj_score):
    minobj = jnp.asarray(min_obj_score, jnp.float32)

    yhat_p = jnp.pad(batch_y_hat, ((0, 0), (0, _NW * _PPW - _NP), (0, 0)))
    yhat_r = yhat_p.reshape(_B, _NW, _PPW, 4).transpose(1, 0, 3, 2)

    obj_pad = jnp.broadcast_to(minobj, (_B, _NW * _PPW - _NP))
    obj_p = jnp.concatenate([batch_obj_scores, obj_pad], axis=1)
    obj_r = obj_p.reshape(_B, _NW, _PPW).transpose(1, 0, 2)

    gt_r = jnp.pad(batch_y.transpose(0, 2, 1),
                   ((0, 0), (0, 0), (0, _NGT_PAD - _NGT))).reshape(-1)

    minobj_vec = jnp.full((16,), minobj, jnp.float32)

    partials = _sc_match(yhat_r, obj_r, gt_r, minobj_vec)
    out = _tc_combine(batch_obj_scores, minobj.reshape(1, 1), partials)
    return out[0, 0]


# speedup vs baseline: 4.8196x; 1.4314x over previous
import functools

import jax
import jax.numpy as jnp
from jax import lax
from jax.experimental import pallas as pl
from jax.experimental.pallas import tpu as pltpu
from jax.experimental.pallas import tpu_sc as plsc

_B = 8
_NP = 20000
_NW = 32
_PPW = 640
_NG = _PPW // 16
_NGT = 100
_NGT_PAD = 112


def _sc_body(yhat_hbm, obj_hbm, gt_hbm, minobj_hbm, out_hbm,
             yhat_v, obj_v, gt_v, minobj_v, out_v):
    c = lax.axis_index("c")
    s = lax.axis_index("s")
    wid = s * 2 + c

    pltpu.sync_copy(yhat_hbm.at[wid], yhat_v)
    pltpu.sync_copy(obj_hbm.at[wid], obj_v)
    pltpu.sync_copy(gt_hbm, gt_v)
    pltpu.sync_copy(minobj_hbm, minobj_v)
    minobj = minobj_v[...]

    zeros_i = jnp.zeros((16,), jnp.int32)
    iota16 = lax.iota(jnp.int32, 16)

    def batch_body(b, carry):
        gbase = b * (4 * _NGT_PAD)
        gidx0 = jnp.full((16,), gbase, jnp.int32)

        def group_body(g, accs):
            loc_acc, m_acc, xm_acc = accs
            sl = pl.ds(g * 16, 16)
            cx = yhat_v[b, 0, sl]
            cy = yhat_v[b, 1, sl]
            w = yhat_v[b, 2, sl]
            h = yhat_v[b, 3, sl]
            ob = obj_v[b, sl]
            hw = w * 0.5
            hh = h * 0.5
            l1 = cx - hw
            r1 = cx + hw
            t1 = cy - hh
            b1 = cy + hh
            a1 = (r1 - l1) * (b1 - t1)
            keep = ob > minobj

            def jg_body(jg, st):
                bi, bu, bj = st
                jo = jg * 16
                gcxv = gt_v[pl.ds(gbase + jo, 16)]
                gcyv = gt_v[pl.ds(gbase + _NGT_PAD + jo, 16)]
                gwv = gt_v[pl.ds(gbase + 2 * _NGT_PAD + jo, 16)]
                ghv = gt_v[pl.ds(gbase + 3 * _NGT_PAD + jo, 16)]
                l2v = gcxv - gwv * 0.5
                r2v = gcxv + gwv * 0.5
                t2v = gcyv - ghv * 0.5
                b2v = gcyv + ghv * 0.5
                a2v = (r2v - l2v) * (b2v - t2v)
                nodes = []
                for je in range(16):
                    l2 = l2v[je]
                    r2 = r2v[je]
                    t2 = t2v[je]
                    b2 = b2v[je]
                    a2 = a2v[je]
                    iw = jnp.maximum(jnp.minimum(r1, r2) - jnp.maximum(l1, l2), 0.0)
                    ih = jnp.maximum(jnp.minimum(b1, b2) - jnp.maximum(t1, t2), 0.0)
                    inter = iw * ih
                    union = (a1 + a2) - inter
                    nodes.append((inter, union, je))
                while len(nodes) > 1:
                    nxt = []
                    for k in range(0, len(nodes), 2):
                        ia, ua, pa = nodes[k]
                        ib, ub, pb = nodes[k + 1]
                        bb = ib * ua > ia * ub
                        nxt.append((jnp.where(bb, ib, ia),
                                    jnp.where(bb, ub, ua),
                                    jnp.where(bb, pb, pa)))
                    nodes = nxt
                gi_, gu_, gp_ = nodes[0]
                better = gi_ * bu > bi * gu_
                bi = jnp.where(better, gi_, bi)
                bu = jnp.where(better, gu_, bu)
                bj = jnp.where(better, jo + gp_, bj)
                return bi, bu, bj

            bi0 = jnp.zeros((16,), jnp.float32)
            bu0 = jnp.ones((16,), jnp.float32)
            bi, bu, bj = lax.fori_loop(0, _NGT_PAD // 16, jg_body,
                                       (bi0, bu0, zeros_i))

            matched = keep & (bi + bi > bu)
            gi = gidx0 + bj
            gcx = plsc.load_gather(gt_v, [gi])
            gcy = plsc.load_gather(gt_v, [gi + _NGT_PAD])
            gw = plsc.load_gather(gt_v, [gi + 2 * _NGT_PAD])
            gh = plsc.load_gather(gt_v, [gi + 3 * _NGT_PAD])
            dcx = cx - gcx
            dcy = cy - gcy
            dw = w - gw
            dh = h - gh
            d = dcx * dcx + dcy * dcy + dw * dw + dh * dh
            zf = jnp.zeros((16,), jnp.float32)
            loc_acc = loc_acc + jnp.where(matched, d, zf)
            m_acc = m_acc + jnp.where(matched, 1.0, 0.0)
            xm_acc = xm_acc + jnp.where(matched, ob, zf)
            return loc_acc, m_acc, xm_acc

        z = jnp.zeros((16,), jnp.float32)
        loc_acc, m_acc, xm_acc = lax.fori_loop(0, _NG, group_body, (z, z, z))
        obase = b * 48
        plsc.store_scatter(out_v, [obase + iota16], loc_acc)
        plsc.store_scatter(out_v, [obase + 16 + iota16], m_acc)
        plsc.store_scatter(out_v, [obase + 32 + iota16], xm_acc)
        return carry
    lax.fori_loop(0, _B, batch_body, 0)

    pltpu.sync_copy(out_v, out_hbm.at[wid])


_sc_match = pl.kernel(
    _sc_body,
    out_type=jax.ShapeDtypeStruct((_NW, _B * 3 * 16), jnp.float32),
    mesh=plsc.VectorSubcoreMesh(core_axis_name="c", subcore_axis_name="s"),
    compiler_params=pltpu.CompilerParams(needs_layout_passes=False),
    scratch_types=[
        pltpu.VMEM((_B, 4, _PPW), jnp.float32),
        pltpu.VMEM((_B, _PPW), jnp.float32),
        pltpu.VMEM((_B * 4 * _NGT_PAD,), jnp.float32),
        pltpu.VMEM((16,), jnp.float32),
        pltpu.VMEM((_B * 3 * 16,), jnp.float32),
    ],
)


def _tc_body(obj_ref, minobj_ref, part_ref, out_ref):
    x = obj_ref[...]
    minobj = minobj_ref[0, 0]
    keep = (x > minobj).astype(jnp.float32)
    k_b = jnp.sum(keep, axis=1)
    bce = jnp.maximum(x, 0.0) + jnp.log1p(jnp.exp(-jnp.abs(x)))
    a_b = jnp.sum(keep * bce, axis=1)
    part = part_ref[...].reshape(_NW, _B, 3, 16)
    sums = jnp.sum(jnp.sum(part, axis=3), axis=0)
    loc_sum = sums[:, 0]
    m = sums[:, 1]
    xm = sums[:, 2]
    loc = jnp.where(m > 0, loc_sum / (4.0 * jnp.maximum(m, 1.0)), 0.0)
    obj = (a_b - xm) / k_b
    pen = 0.1 * (k_b - m)
    total = jnp.sum(loc + obj + pen) / _B
    out_ref[...] = jnp.full((1, 1), total, jnp.float32)


_tc_combine = pl.pallas_call(
    _tc_body,
    out_shape=jax.ShapeDtypeStruct((1, 1), jnp.float32),
)


def kernel(batch_y_hat, batch_y, batch_obj_scores, min_obj_score):
    minobj = jnp.asarray(min_obj_score, jnp.float32)

    yhat_p = jnp.pad(batch_y_hat, ((0, 0), (0, _NW * _PPW - _NP), (0, 0)))
    yhat_r = yhat_p.reshape(_B, _NW, _PPW, 4).transpose(1, 0, 3, 2)

    obj_pad = jnp.broadcast_to(minobj, (_B, _NW * _PPW - _NP))
    obj_p = jnp.concatenate([batch_obj_scores, obj_pad], axis=1)
    obj_r = obj_p.reshape(_B, _NW, _PPW).transpose(1, 0, 2)

    gt_r = jnp.pad(batch_y.transpose(0, 2, 1),
                   ((0, 0), (0, 0), (0, _NGT_PAD - _NGT))).reshape(-1)

    minobj_vec = jnp.full((16,), minobj, jnp.float32)

    partials = _sc_match(yhat_r, obj_r, gt_r, minobj_vec)
    out = _tc_combine(batch_obj_scores, minobj.reshape(1, 1), partials)
    return out[0, 0]


# speedup vs baseline: 7.7727x; 1.6127x over previous
import functools

import jax
import jax.numpy as jnp
from jax import lax
from jax.experimental import pallas as pl
from jax.experimental.pallas import tpu as pltpu
from jax.experimental.pallas import tpu_sc as plsc

_B = 8
_NP = 20000
_NW = 32
_PPW = 640
_NG = _PPW // 16
_NGT = 100
_NGT_PAD = 112


def _sc_body(yhat_hbm, obj_hbm, gt_hbm, minobj_hbm, out_hbm,
             yhat_v, obj_v, gt_v, cbuf, minobj_v, out_v):
    c = lax.axis_index("c")
    s = lax.axis_index("s")
    wid = s * 2 + c

    pltpu.sync_copy(yhat_hbm.at[wid], yhat_v)
    pltpu.sync_copy(obj_hbm.at[wid], obj_v)
    pltpu.sync_copy(gt_hbm, gt_v)
    pltpu.sync_copy(minobj_hbm, minobj_v)
    minobj = minobj_v[...]

    zeros_i = jnp.zeros((16,), jnp.int32)
    iota16 = lax.iota(jnp.int32, 16)

    def batch_body(b, carry):
        gbase = b * (4 * _NGT_PAD)
        gidx0 = jnp.full((16,), gbase, jnp.int32)

        def compact_g(g, cnt):
            sl = pl.ds(g * 16, 16)
            cx = yhat_v[b, 0, sl]
            cy = yhat_v[b, 1, sl]
            w = yhat_v[b, 2, sl]
            h = yhat_v[b, 3, sl]
            ob = obj_v[b, sl]
            keep = ob > minobj
            pos = plsc.cumsum(keep.astype(jnp.int32))
            idx = (cnt - 1) + pos
            plsc.store_scatter(cbuf, [idx], cx, mask=keep)
            plsc.store_scatter(cbuf, [idx + _PPW], cy, mask=keep)
            plsc.store_scatter(cbuf, [idx + 2 * _PPW], w, mask=keep)
            plsc.store_scatter(cbuf, [idx + 3 * _PPW], h, mask=keep)
            plsc.store_scatter(cbuf, [idx + 4 * _PPW], ob, mask=keep)
            return cnt + pos[15]

        cnt = lax.fori_loop(0, _NG, compact_g, jnp.int32(0))
        ng2 = (cnt + 15) // 16

        def group_body(g, accs):
            loc_acc, m_acc, xm_acc = accs
            sl = pl.ds(g * 16, 16)
            cx = cbuf[sl]
            cy = cbuf[pl.ds(_PPW + g * 16, 16)]
            w = cbuf[pl.ds(2 * _PPW + g * 16, 16)]
            h = cbuf[pl.ds(3 * _PPW + g * 16, 16)]
            ob = cbuf[pl.ds(4 * _PPW + g * 16, 16)]
            hw = w * 0.5
            hh = h * 0.5
            l1 = cx - hw
            r1 = cx + hw
            t1 = cy - hh
            b1 = cy + hh
            a1 = (r1 - l1) * (b1 - t1)
            valid = iota16 < (cnt - g * 16)

            def jg_body(jg, st):
                bi, bu, bj = st
                jo = jg * 16
                gcxv = gt_v[pl.ds(gbase + jo, 16)]
                gcyv = gt_v[pl.ds(gbase + _NGT_PAD + jo, 16)]
                gwv = gt_v[pl.ds(gbase + 2 * _NGT_PAD + jo, 16)]
                ghv = gt_v[pl.ds(gbase + 3 * _NGT_PAD + jo, 16)]
                l2v = gcxv - gwv * 0.5
                r2v = gcxv + gwv * 0.5
                t2v = gcyv - ghv * 0.5
                b2v = gcyv + ghv * 0.5
                a2v = (r2v - l2v) * (b2v - t2v)
                def leaf(je):
                    l2 = l2v[je]
                    r2 = r2v[je]
                    t2 = t2v[je]
                    b2 = b2v[je]
                    a2 = a2v[je]
                    iw = jnp.maximum(jnp.minimum(r1, r2) - jnp.maximum(l1, l2), 0.0)
                    ih = jnp.maximum(jnp.minimum(b1, b2) - jnp.maximum(t1, t2), 0.0)
                    inter = iw * ih
                    union = (a1 + a2) - inter
                    return inter, union, je

                nodes = []
                for k in range(8):
                    ia, ua, pa = leaf(2 * k)
                    ib, ub, pb = leaf(2 * k + 1)
                    bb = ib * ua > ia * ub
                    nodes.append((jnp.where(bb, ib, ia),
                                  jnp.where(bb, ub, ua),
                                  jnp.where(bb, pb, pa)))
                while len(nodes) > 1:
                    nxt = []
                    for k in range(0, len(nodes), 2):
                        ia, ua, pa = nodes[k]
                        ib, ub, pb = nodes[k + 1]
                        bb = ib * ua > ia * ub
                        nxt.append((jnp.where(bb, ib, ia),
                                    jnp.where(bb, ub, ua),
                                    jnp.where(bb, pb, pa)))
                    nodes = nxt
                gi_, gu_, gp_ = nodes[0]
                better = gi_ * bu > bi * gu_
                bi = jnp.where(better, gi_, bi)
                bu = jnp.where(better, gu_, bu)
                bj = jnp.where(better, jo + gp_, bj)
                return bi, bu, bj

            bi0 = jnp.zeros((16,), jnp.float32)
            bu0 = jnp.ones((16,), jnp.float32)
            bi, bu, bj = lax.fori_loop(0, _NGT_PAD // 16, jg_body,
                                       (bi0, bu0, zeros_i))

            matched = valid & (bi + bi > bu)
            gi = gidx0 + bj
            gcx = plsc.load_gather(gt_v, [gi])
            gcy = plsc.load_gather(gt_v, [gi + _NGT_PAD])
            gw = plsc.load_gather(gt_v, [gi + 2 * _NGT_PAD])
            gh = plsc.load_gather(gt_v, [gi + 3 * _NGT_PAD])
            dcx = cx - gcx
            dcy = cy - gcy
            dw = w - gw
            dh = h - gh
            d = dcx * dcx + dcy * dcy + dw * dw + dh * dh
            zf = jnp.zeros((16,), jnp.float32)
            loc_acc = loc_acc + jnp.where(matched, d, zf)
            m_acc = m_acc + jnp.where(matched, 1.0, 0.0)
            xm_acc = xm_acc + jnp.where(matched, ob, zf)
            return loc_acc, m_acc, xm_acc

        z = jnp.zeros((16,), jnp.float32)
        loc_acc, m_acc, xm_acc = lax.fori_loop(0, ng2, group_body, (z, z, z))
        obase = b * 48
        plsc.store_scatter(out_v, [obase + iota16], loc_acc)
        plsc.store_scatter(out_v, [obase + 16 + iota16], m_acc)
        plsc.store_scatter(out_v, [obase + 32 + iota16], xm_acc)
        return carry
    lax.fori_loop(0, _B, batch_body, 0)

    pltpu.sync_copy(out_v, out_hbm.at[wid])


_sc_match = pl.kernel(
    _sc_body,
    out_type=jax.ShapeDtypeStruct((_NW, _B * 3 * 16), jnp.float32),
    mesh=plsc.VectorSubcoreMesh(core_axis_name="c", subcore_axis_name="s"),
    compiler_params=pltpu.CompilerParams(needs_layout_passes=False),
    scratch_types=[
        pltpu.VMEM((_B, 4, _PPW), jnp.float32),
        pltpu.VMEM((_B, _PPW), jnp.float32),
        pltpu.VMEM((_B * 4 * _NGT_PAD,), jnp.float32),
        pltpu.VMEM((5 * _PPW,), jnp.float32),
        pltpu.VMEM((16,), jnp.float32),
        pltpu.VMEM((_B * 3 * 16,), jnp.float32),
    ],
)


def _tc_body(obj_ref, minobj_ref, part_ref, out_ref):
    x = obj_ref[...]
    minobj = minobj_ref[0, 0]
    keep = (x > minobj).astype(jnp.float32)
    k_b = jnp.sum(keep, axis=1)
    bce = jnp.maximum(x, 0.0) + jnp.log1p(jnp.exp(-jnp.abs(x)))
    a_b = jnp.sum(keep * bce, axis=1)
    part = part_ref[...].reshape(_NW, _B, 3, 16)
    sums = jnp.sum(jnp.sum(part, axis=3), axis=0)
    loc_sum = sums[:, 0]
    m = sums[:, 1]
    xm = sums[:, 2]
    loc = jnp.where(m > 0, loc_sum / (4.0 * jnp.maximum(m, 1.0)), 0.0)
    obj = (a_b - xm) / k_b
    pen = 0.1 * (k_b - m)
    total = jnp.sum(loc + obj + pen) / _B
    out_ref[...] = jnp.full((1, 1), total, jnp.float32)


_tc_combine = pl.pallas_call(
    _tc_body,
    out_shape=jax.ShapeDtypeStruct((1, 1), jnp.float32),
)


def kernel(batch_y_hat, batch_y, batch_obj_scores, min_obj_score):
    minobj = jnp.asarray(min_obj_score, jnp.float32)

    yhat_p = jnp.pad(batch_y_hat, ((0, 0), (0, _NW * _PPW - _NP), (0, 0)))
    yhat_r = yhat_p.reshape(_B, _NW, _PPW, 4).transpose(1, 0, 3, 2)

    obj_pad = jnp.broadcast_to(minobj, (_B, _NW * _PPW - _NP))
    obj_p = jnp.concatenate([batch_obj_scores, obj_pad], axis=1)
    obj_r = obj_p.reshape(_B, _NW, _PPW).transpose(1, 0, 2)

    gt_r = jnp.pad(batch_y.transpose(0, 2, 1),
                   ((0, 0), (0, 0), (0, _NGT_PAD - _NGT))).reshape(-1)

    minobj_vec = jnp.full((16,), minobj, jnp.float32)

    partials = _sc_match(yhat_r, obj_r, gt_r, minobj_vec)
    out = _tc_combine(batch_obj_scores, minobj.reshape(1, 1), partials)
    return out[0, 0]
